# Initial kernel scaffold; baseline (speedup 1.0000x reference)
import jax
import jax.numpy as jnp
from jax.experimental import pallas as pl


def kernel(inputs, edge_index, w, a1_w, a1_b, bn1_g, bn1_b, a2_w, a2_b, bn2_g, bn2_b, a3_w, a3_b):
    raise NotImplementedError("write your pallas kernel here")


# speedup vs baseline: 2.0918x; 2.0918x over previous
import functools

import jax
import jax.numpy as jnp
from jax import lax
from jax.experimental import pallas as pl
from jax.experimental.pallas import tpu as pltpu
from jax.experimental.pallas import tpu_sc as plsc

N = 10000
E = 320000
D = 128
D1 = 64
D2 = 32

NC = 2
NS = 16
NW = NC * NS

E_PER_W = E // NW
E_PER_CW = E // NC // NS
CHUNK = 80
N_CHUNKS = E_PER_W // CHUNK
N_PAD = 10240
ROWS_PER_TILE = N_PAD // NS

BE = 2560
GE = E // BE

_mesh = plsc.VectorSubcoreMesh(core_axis_name="c", subcore_axis_name="s")

_GATHER_DNUMS = lax.GatherDimensionNumbers(
    offset_dims=(), collapsed_slice_dims=(0,), start_index_map=(0,))


def _bcast_lane(vec16, lane):
    idx = jnp.full((16, 1), lane, jnp.int32)
    return lax.gather(vec16, idx, _GATHER_DNUMS, (1,),
                      mode=lax.GatherScatterMode.PROMISE_IN_BOUNDS)


def _mm_h_body(x_ref, w_ref, o_ref):
    o_ref[...] = jnp.dot(x_ref[...], w_ref[...],
                         preferred_element_type=jnp.float32)


_mm_h = pl.pallas_call(
    _mm_h_body,
    out_shape=jax.ShapeDtypeStruct((N, D), jnp.float32),
)


@functools.partial(
    pl.kernel,
    mesh=_mesh,
    out_type=jax.ShapeDtypeStruct((E, D), jnp.float32),
    scratch_types=[
        pltpu.VMEM((CHUNK,), jnp.int32),
        pltpu.VMEM((CHUNK,), jnp.int32),
        pltpu.VMEM((CHUNK, D), jnp.float32),
        pltpu.VMEM((CHUNK, D), jnp.float32),
        pltpu.VMEM((CHUNK, D), jnp.float32),
        pltpu.SemaphoreType.DMA,
        pltpu.SemaphoreType.DMA,
    ],
)
def _sc_edge_diff(h_hbm, src_hbm, dst_hbm, eh_hbm, si, di, hs, hd, ob,
                  sem_a, sem_b):
    cid = lax.axis_index("c")
    sid = lax.axis_index("s")
    wid = sid * NC + cid
    base = wid * E_PER_W

    def chunk(c, carry):
        off = base + c * CHUNK
        pltpu.sync_copy(src_hbm.at[pl.ds(off, CHUNK)], si)
        pltpu.sync_copy(dst_hbm.at[pl.ds(off, CHUNK)], di)
        ca = pltpu.async_copy(h_hbm.at[si], hs, sem_a)
        cb = pltpu.async_copy(h_hbm.at[di], hd, sem_b)
        ca.wait()
        cb.wait()

        def row(r, carry2):
            for j in range(D // 16):
                a = hs[r, pl.ds(j * 16, 16)]
                b = hd[r, pl.ds(j * 16, 16)]
                ob[r, pl.ds(j * 16, 16)] = jnp.abs(a - b)
            return carry2

        lax.fori_loop(0, CHUNK, row, 0)
        pltpu.sync_copy(ob, eh_hbm.at[pl.ds(off, CHUNK)])
        return carry

    lax.fori_loop(0, N_CHUNKS, chunk, 0)


def _stage1_body(eh_ref, a1w_ref, a1b_ref, x1_ref, st_ref):
    x = jnp.dot(eh_ref[...], a1w_ref[...],
                preferred_element_type=jnp.float32) + a1b_ref[...]
    x1_ref[...] = x
    upd = jnp.concatenate(
        [jnp.sum(x, axis=0)[None], jnp.sum(x * x, axis=0)[None],
         jnp.zeros((6, D1), jnp.float32)], axis=0)

    @pl.when(pl.program_id(0) == 0)
    def _():
        st_ref[...] = upd

    @pl.when(pl.program_id(0) > 0)
    def _():
        st_ref[...] = st_ref[...] + upd


_stage1 = pl.pallas_call(
    _stage1_body,
    grid=(GE,),
    in_specs=[
        pl.BlockSpec((BE, D), lambda i: (i, 0)),
        pl.BlockSpec((D, D1), lambda i: (0, 0)),
        pl.BlockSpec((1, D1), lambda i: (0, 0)),
    ],
    out_specs=[
        pl.BlockSpec((BE, D1), lambda i: (i, 0)),
        pl.BlockSpec((8, D1), lambda i: (0, 0)),
    ],
    out_shape=[
        jax.ShapeDtypeStruct((E, D1), jnp.float32),
        jax.ShapeDtypeStruct((8, D1), jnp.float32),
    ],
    compiler_params=pltpu.CompilerParams(dimension_semantics=("arbitrary",)),
)


def _stage2_body(x1_ref, st_ref, g_ref, b_ref, a2w_ref, a2b_ref,
                 x2_ref, st2_ref):
    st = st_ref[...]
    mean = st[0:1, :] * (1.0 / E)
    ex2 = st[1:2, :] * (1.0 / E)
    var = ex2 - mean * mean
    scale = g_ref[...] * lax.rsqrt(var + 1e-5)
    shift = b_ref[...] - mean * scale
    y = x1_ref[...] * scale + shift
    y = jnp.maximum(y, 0.2 * y)
    x2 = jnp.dot(y, a2w_ref[...],
                 preferred_element_type=jnp.float32) + a2b_ref[...]
    x2_ref[...] = x2
    upd = jnp.concatenate(
        [jnp.sum(x2, axis=0)[None], jnp.sum(x2 * x2, axis=0)[None],
         jnp.zeros((6, D2), jnp.float32)], axis=0)

    @pl.when(pl.program_id(0) == 0)
    def _():
        st2_ref[...] = upd

    @pl.when(pl.program_id(0) > 0)
    def _():
        st2_ref[...] = st2_ref[...] + upd


_stage2 = pl.pallas_call(
    _stage2_body,
    grid=(GE,),
    in_specs=[
        pl.BlockSpec((BE, D1), lambda i: (i, 0)),
        pl.BlockSpec((8, D1), lambda i: (0, 0)),
        pl.BlockSpec((1, D1), lambda i: (0, 0)),
        pl.BlockSpec((1, D1), lambda i: (0, 0)),
        pl.BlockSpec((D1, D2), lambda i: (0, 0)),
        pl.BlockSpec((1, D2), lambda i: (0, 0)),
    ],
    out_specs=[
        pl.BlockSpec((BE, D2), lambda i: (i, 0)),
        pl.BlockSpec((8, D2), lambda i: (0, 0)),
    ],
    out_shape=[
        jax.ShapeDtypeStruct((E, D2), jnp.float32),
        jax.ShapeDtypeStruct((8, D2), jnp.float32),
    ],
    compiler_params=pltpu.CompilerParams(dimension_semantics=("arbitrary",)),
)


def _stage3_body(x2_ref, st2_ref, g_ref, b_ref, a3w_ref, a3b_ref,
                 src_ref, dst_ref, e_ref):
    st = st2_ref[...]
    mean = st[0:1, :] * (1.0 / E)
    ex2 = st[1:2, :] * (1.0 / E)
    var = ex2 - mean * mean
    scale = g_ref[...] * lax.rsqrt(var + 1e-5)
    shift = b_ref[...] - mean * scale
    z = x2_ref[...] * scale + shift
    z = jnp.maximum(z, 0.2 * z)
    t = jnp.sum(z * a3w_ref[...], axis=1, keepdims=True) + a3b_ref[...]
    t = jnp.maximum(t, 0.2 * t)
    ev = jnp.exp(-t[:, 0])
    ev = ev + (src_ref[0, 0, :] == dst_ref[0, 0, :]).astype(jnp.float32)
    e_ref[0, 0, :] = ev


_stage3 = pl.pallas_call(
    _stage3_body,
    grid=(GE,),
    in_specs=[
        pl.BlockSpec((BE, D2), lambda i: (i, 0)),
        pl.BlockSpec((8, D2), lambda i: (0, 0)),
        pl.BlockSpec((1, D2), lambda i: (0, 0)),
        pl.BlockSpec((1, D2), lambda i: (0, 0)),
        pl.BlockSpec((1, D2), lambda i: (0, 0)),
        pl.BlockSpec((1, 1), lambda i: (0, 0)),
        pl.BlockSpec((1, 1, BE), lambda i: (i, 0, 0)),
        pl.BlockSpec((1, 1, BE), lambda i: (i, 0, 0)),
    ],
    out_specs=pl.BlockSpec((1, 1, BE), lambda i: (i, 0, 0)),
    out_shape=jax.ShapeDtypeStruct((GE, 1, BE), jnp.float32),
    compiler_params=pltpu.CompilerParams(dimension_semantics=("arbitrary",)),
)


@functools.partial(
    pl.kernel,
    mesh=_mesh,
    out_type=[
        jax.ShapeDtypeStruct((NC, N_PAD, D), jnp.float32),
        jax.ShapeDtypeStruct((NC, N_PAD), jnp.float32),
    ],
    scratch_types=[
        pltpu.VMEM((CHUNK,), jnp.int32),
        pltpu.VMEM((CHUNK,), jnp.int32),
        pltpu.VMEM((CHUNK,), jnp.float32),
        pltpu.VMEM((CHUNK, D), jnp.float32),
        pltpu.VMEM((CHUNK, D), jnp.float32),
        pltpu.VMEM((128, D), jnp.float32),
        pltpu.VMEM((ROWS_PER_TILE,), jnp.float32),
        pltpu.VMEM_SHARED((N_PAD, D), jnp.float32),
        pltpu.VMEM_SHARED((N_PAD,), jnp.float32),
        pltpu.SemaphoreType.DMA,
    ],
)
def _sc_scatter(h_hbm, src_hbm, dst_hbm, e_hbm, hp_hbm, rs_hbm,
                is_, id_, ev_, rows, sb, zb, zb1, acc, acc_r, sem):
    cid = lax.axis_index("c")
    sid = lax.axis_index("s")

    def zrow(r, c2):
        for j in range(D // 16):
            zb[r, pl.ds(j * 16, 16)] = jnp.zeros((16,), jnp.float32)
        return c2

    lax.fori_loop(0, 128, zrow, 0)

    def zrow1(r, c2):
        zb1[pl.ds(r * 16, 16)] = jnp.zeros((16,), jnp.float32)
        return c2

    lax.fori_loop(0, ROWS_PER_TILE // 16, zrow1, 0)
    r0 = sid * ROWS_PER_TILE
    for k in range(ROWS_PER_TILE // 128):
        pltpu.sync_copy(zb, acc.at[pl.ds(r0 + k * 128, 128)])
    pltpu.sync_copy(zb1, acc_r.at[pl.ds(r0, ROWS_PER_TILE)])
    plsc.subcore_barrier()

    base = cid * (E // NC) + sid * E_PER_CW

    def chunk(c, carry):
        off = base + c * CHUNK
        pltpu.sync_copy(src_hbm.at[pl.ds(off, CHUNK)], is_)
        pltpu.sync_copy(dst_hbm.at[pl.ds(off, CHUNK)], id_)
        pltpu.sync_copy(e_hbm.at[pl.ds(off, CHUNK)], ev_)
        pltpu.async_copy(h_hbm.at[id_], rows, sem).wait()
        for g in range(CHUNK // 16):
            eg = ev_[pl.ds(g * 16, 16)]
            for l in range(16):
                r = g * 16 + l
                bc = _bcast_lane(eg, l)
                for j in range(D // 16):
                    sb[r, pl.ds(j * 16, 16)] = rows[r, pl.ds(j * 16, 16)] * bc
        pltpu.sync_copy(sb, acc.at[is_], add=True)
        pltpu.sync_copy(ev_, acc_r.at[is_], add=True)
        return carry

    lax.fori_loop(0, E_PER_CW // CHUNK, chunk, 0)
    plsc.subcore_barrier()
    pltpu.sync_copy(acc.at[pl.ds(r0, ROWS_PER_TILE)],
                    hp_hbm.at[cid, pl.ds(r0, ROWS_PER_TILE)])
    pltpu.sync_copy(acc_r.at[pl.ds(r0, ROWS_PER_TILE)],
                    rs_hbm.at[cid, pl.ds(r0, ROWS_PER_TILE)])


def _fin_body(hp_ref, rs_ref, o_ref):
    hp = hp_ref[0] + hp_ref[1]
    rs = rs_ref[0] + rs_ref[1]
    rs = rs + (rs == 0.0).astype(jnp.float32)
    v = hp / rs
    o_ref[...] = jnp.maximum(v, 0.2 * v)


_fin = pl.pallas_call(
    _fin_body,
    out_shape=jax.ShapeDtypeStruct((N_PAD, D), jnp.float32),
)


def kernel(inputs, edge_index, w, a1_w, a1_b, bn1_g, bn1_b,
           a2_w, a2_b, bn2_g, bn2_b, a3_w, a3_b):
    ei = edge_index.astype(jnp.int32)
    src = ei[0]
    dst = ei[1]
    h = _mm_h(inputs, w)
    eh = _sc_edge_diff(h, src, dst)
    x1, st1 = _stage1(eh, a1_w, a1_b.reshape(1, D1))
    x2, st2 = _stage2(x1, st1, bn1_g.reshape(1, D1), bn1_b.reshape(1, D1),
                      a2_w, a2_b.reshape(1, D2))
    e = _stage3(x2, st2, bn2_g.reshape(1, D2), bn2_b.reshape(1, D2),
                a3_w.reshape(1, D2), a3_b.reshape(1, 1),
                src.reshape(GE, 1, BE), dst.reshape(GE, 1, BE))
    hp2, rs2 = _sc_scatter(h, src, dst, e.reshape(E))
    return _fin(hp2, rs2.reshape(NC, N_PAD, 1))[:N]


# speedup vs baseline: 3.2432x; 1.5504x over previous
import functools

import jax
import jax.numpy as jnp
from jax import lax
from jax.experimental import pallas as pl
from jax.experimental.pallas import tpu as pltpu
from jax.experimental.pallas import tpu_sc as plsc

N = 10000
E = 320000
D = 128
D1 = 64
D2 = 32

NC = 2
NS = 16
NW = NC * NS

E_PER_W = E // NW
E_PER_CW = E // NC // NS
CHUNK = 80
N_CHUNKS = E_PER_W // CHUNK
N_PAD = 10240
ROWS_PER_TILE = N_PAD // NS

BE = 2560
GE = E // BE

_mesh = plsc.VectorSubcoreMesh(core_axis_name="c", subcore_axis_name="s")

_GATHER_DNUMS = lax.GatherDimensionNumbers(
    offset_dims=(), collapsed_slice_dims=(0,), start_index_map=(0,))


def _bcast_lane(vec16, lane):
    idx = jnp.full((16, 1), lane, jnp.int32)
    return lax.gather(vec16, idx, _GATHER_DNUMS, (1,),
                      mode=lax.GatherScatterMode.PROMISE_IN_BOUNDS)


def _mm_h_body(x_ref, w_ref, o_ref):
    o_ref[...] = jnp.dot(x_ref[...], w_ref[...],
                         preferred_element_type=jnp.float32)


_mm_h = pl.pallas_call(
    _mm_h_body,
    out_shape=jax.ShapeDtypeStruct((N, D), jnp.float32),
)


@functools.partial(
    pl.kernel,
    mesh=_mesh,
    out_type=jax.ShapeDtypeStruct((E, D), jnp.float32),
    scratch_types=[
        pltpu.VMEM((N_CHUNKS, CHUNK), jnp.int32),
        pltpu.VMEM((N_CHUNKS, CHUNK), jnp.int32),
        pltpu.VMEM((2, CHUNK, D), jnp.float32),
        pltpu.VMEM((2, CHUNK, D), jnp.float32),
        pltpu.VMEM((2, CHUNK, D), jnp.float32),
        pltpu.SemaphoreType.DMA,
        pltpu.SemaphoreType.DMA,
        pltpu.SemaphoreType.DMA,
        pltpu.SemaphoreType.DMA,
        pltpu.SemaphoreType.DMA,
        pltpu.SemaphoreType.DMA,
    ],
)
def _sc_edge_diff(h_hbm, src3_hbm, dst3_hbm, eh_hbm, si2, di2, hs2, hd2, ob2,
                  sga0, sga1, sgb0, sgb1, sst0, sst1):
    cid = lax.axis_index("c")
    sid = lax.axis_index("s")
    wid = sid * NC + cid
    base = wid * E_PER_W
    sga = (sga0, sga1)
    sgb = (sgb0, sgb1)
    sst = (sst0, sst1)

    pltpu.sync_copy(src3_hbm.at[wid], si2)
    pltpu.sync_copy(dst3_hbm.at[wid], di2)
    for b in range(2):
        pltpu.async_copy(h_hbm.at[si2.at[b]], hs2.at[b], sga[b])
        pltpu.async_copy(h_hbm.at[di2.at[b]], hd2.at[b], sgb[b])

    def compute(b, c):
        def row(r, carry2):
            for j in range(D // 16):
                a = hs2[b, r, pl.ds(j * 16, 16)]
                bb = hd2[b, r, pl.ds(j * 16, 16)]
                ob2[b, r, pl.ds(j * 16, 16)] = jnp.abs(a - bb)
            return carry2

        lax.fori_loop(0, CHUNK, row, 0)

    def process(b, c):
        @pl.when(c >= 2)
        def _():
            pltpu.make_async_copy(
                ob2.at[b], eh_hbm.at[pl.ds(0, CHUNK)], sst[b]).wait()
        pltpu.make_async_copy(h_hbm.at[si2.at[b]], hs2.at[b], sga[b]).wait()
        pltpu.make_async_copy(h_hbm.at[di2.at[b]], hd2.at[b], sgb[b]).wait()
        compute(b, c)
        pltpu.async_copy(ob2.at[b], eh_hbm.at[pl.ds(base + c * CHUNK, CHUNK)],
                         sst[b])

        @pl.when(c + 2 < N_CHUNKS)
        def _():
            pltpu.async_copy(h_hbm.at[si2.at[c + 2]], hs2.at[b], sga[b])
            pltpu.async_copy(h_hbm.at[di2.at[c + 2]], hd2.at[b], sgb[b])

    def step(i, carry):
        for b in range(2):
            process(b, 2 * i + b)
        return carry

    lax.fori_loop(0, (N_CHUNKS - 1) // 2, step, 0)
    process(0, N_CHUNKS - 1)
    for b in range(2):
        pltpu.make_async_copy(
            ob2.at[b], eh_hbm.at[pl.ds(0, CHUNK)], sst[b]).wait()


def _stage1_body(eh_ref, a1w_ref, a1b_ref, x1_ref, st_ref):
    x = jnp.dot(eh_ref[...], a1w_ref[...],
                preferred_element_type=jnp.float32) + a1b_ref[...]
    x1_ref[...] = x
    upd = jnp.concatenate(
        [jnp.sum(x, axis=0)[None], jnp.sum(x * x, axis=0)[None],
         jnp.zeros((6, D1), jnp.float32)], axis=0)

    @pl.when(pl.program_id(0) == 0)
    def _():
        st_ref[...] = upd

    @pl.when(pl.program_id(0) > 0)
    def _():
        st_ref[...] = st_ref[...] + upd


_stage1 = pl.pallas_call(
    _stage1_body,
    grid=(GE,),
    in_specs=[
        pl.BlockSpec((BE, D), lambda i: (i, 0)),
        pl.BlockSpec((D, D1), lambda i: (0, 0)),
        pl.BlockSpec((1, D1), lambda i: (0, 0)),
    ],
    out_specs=[
        pl.BlockSpec((BE, D1), lambda i: (i, 0)),
        pl.BlockSpec((8, D1), lambda i: (0, 0)),
    ],
    out_shape=[
        jax.ShapeDtypeStruct((E, D1), jnp.float32),
        jax.ShapeDtypeStruct((8, D1), jnp.float32),
    ],
    compiler_params=pltpu.CompilerParams(dimension_semantics=("arbitrary",)),
)


def _stage2_body(x1_ref, st_ref, g_ref, b_ref, a2w_ref, a2b_ref,
                 x2_ref, st2_ref):
    st = st_ref[...]
    mean = st[0:1, :] * (1.0 / E)
    ex2 = st[1:2, :] * (1.0 / E)
    var = ex2 - mean * mean
    scale = g_ref[...] * lax.rsqrt(var + 1e-5)
    shift = b_ref[...] - mean * scale
    y = x1_ref[...] * scale + shift
    y = jnp.maximum(y, 0.2 * y)
    x2 = jnp.dot(y, a2w_ref[...],
                 preferred_element_type=jnp.float32) + a2b_ref[...]
    x2_ref[...] = x2
    upd = jnp.concatenate(
        [jnp.sum(x2, axis=0)[None], jnp.sum(x2 * x2, axis=0)[None],
         jnp.zeros((6, D2), jnp.float32)], axis=0)

    @pl.when(pl.program_id(0) == 0)
    def _():
        st2_ref[...] = upd

    @pl.when(pl.program_id(0) > 0)
    def _():
        st2_ref[...] = st2_ref[...] + upd


_stage2 = pl.pallas_call(
    _stage2_body,
    grid=(GE,),
    in_specs=[
        pl.BlockSpec((BE, D1), lambda i: (i, 0)),
        pl.BlockSpec((8, D1), lambda i: (0, 0)),
        pl.BlockSpec((1, D1), lambda i: (0, 0)),
        pl.BlockSpec((1, D1), lambda i: (0, 0)),
        pl.BlockSpec((D1, D2), lambda i: (0, 0)),
        pl.BlockSpec((1, D2), lambda i: (0, 0)),
    ],
    out_specs=[
        pl.BlockSpec((BE, D2), lambda i: (i, 0)),
        pl.BlockSpec((8, D2), lambda i: (0, 0)),
    ],
    out_shape=[
        jax.ShapeDtypeStruct((E, D2), jnp.float32),
        jax.ShapeDtypeStruct((8, D2), jnp.float32),
    ],
    compiler_params=pltpu.CompilerParams(dimension_semantics=("arbitrary",)),
)


def _stage3_body(x2_ref, st2_ref, g_ref, b_ref, a3w_ref, a3b_ref,
                 src_ref, dst_ref, e_ref):
    st = st2_ref[...]
    mean = st[0:1, :] * (1.0 / E)
    ex2 = st[1:2, :] * (1.0 / E)
    var = ex2 - mean * mean
    scale = g_ref[...] * lax.rsqrt(var + 1e-5)
    shift = b_ref[...] - mean * scale
    z = x2_ref[...] * scale + shift
    z = jnp.maximum(z, 0.2 * z)
    t = jnp.sum(z * a3w_ref[...], axis=1, keepdims=True) + a3b_ref[...]
    t = jnp.maximum(t, 0.2 * t)
    ev = jnp.exp(-t[:, 0])
    ev = ev + (src_ref[0, 0, :] == dst_ref[0, 0, :]).astype(jnp.float32)
    e_ref[0, 0, :] = ev


_stage3 = pl.pallas_call(
    _stage3_body,
    grid=(GE,),
    in_specs=[
        pl.BlockSpec((BE, D2), lambda i: (i, 0)),
        pl.BlockSpec((8, D2), lambda i: (0, 0)),
        pl.BlockSpec((1, D2), lambda i: (0, 0)),
        pl.BlockSpec((1, D2), lambda i: (0, 0)),
        pl.BlockSpec((1, D2), lambda i: (0, 0)),
        pl.BlockSpec((1, 1), lambda i: (0, 0)),
        pl.BlockSpec((1, 1, BE), lambda i: (i, 0, 0)),
        pl.BlockSpec((1, 1, BE), lambda i: (i, 0, 0)),
    ],
    out_specs=pl.BlockSpec((1, 1, BE), lambda i: (i, 0, 0)),
    out_shape=jax.ShapeDtypeStruct((GE, 1, BE), jnp.float32),
    compiler_params=pltpu.CompilerParams(dimension_semantics=("arbitrary",)),
)


@functools.partial(
    pl.kernel,
    mesh=_mesh,
    out_type=[
        jax.ShapeDtypeStruct((NC, N_PAD, D), jnp.float32),
        jax.ShapeDtypeStruct((NC, N_PAD), jnp.float32),
    ],
    scratch_types=[
        pltpu.VMEM((2, CHUNK), jnp.int32),
        pltpu.VMEM((2, CHUNK), jnp.int32),
        pltpu.VMEM((2, CHUNK), jnp.float32),
        pltpu.VMEM((2, CHUNK, D), jnp.float32),
        pltpu.VMEM((CHUNK, D), jnp.float32),
        pltpu.VMEM((ROWS_PER_TILE,), jnp.float32),
        pltpu.VMEM_SHARED((N_PAD, D), jnp.float32),
        pltpu.VMEM_SHARED((N_PAD,), jnp.float32),
        pltpu.SemaphoreType.DMA,
        pltpu.SemaphoreType.DMA,
        pltpu.SemaphoreType.DMA,
        pltpu.SemaphoreType.DMA,
        pltpu.SemaphoreType.DMA,
        pltpu.SemaphoreType.DMA,
        pltpu.SemaphoreType.DMA,
        pltpu.SemaphoreType.DMA,
    ],
)
def _sc_scatter(h_hbm, src_hbm, dst_hbm, e_hbm, hp_hbm, rs_hbm,
                is_, id_, ev_, rows2, sb, zb1, acc, acc_r,
                sg0, sg1, sid0, sid1, sis0, sis1, sev0, sev1):
    cid = lax.axis_index("c")
    sid = lax.axis_index("s")
    sg = (sg0, sg1)
    sdi = (sid0, sid1)
    sis = (sis0, sis1)
    sev = (sev0, sev1)

    def zrow(r, c2):
        for j in range(D // 16):
            sb[r, pl.ds(j * 16, 16)] = jnp.zeros((16,), jnp.float32)
        return c2

    lax.fori_loop(0, CHUNK, zrow, 0)

    def zrow1(r, c2):
        zb1[pl.ds(r * 16, 16)] = jnp.zeros((16,), jnp.float32)
        return c2

    lax.fori_loop(0, ROWS_PER_TILE // 16, zrow1, 0)
    r0 = sid * ROWS_PER_TILE
    for k in range(ROWS_PER_TILE // CHUNK):
        pltpu.sync_copy(sb, acc.at[pl.ds(r0 + k * CHUNK, CHUNK)])
    pltpu.sync_copy(zb1, acc_r.at[pl.ds(r0, ROWS_PER_TILE)])
    plsc.subcore_barrier()

    base = (cid * NS + sid) * E_PER_CW
    for b in range(2):
        off = base + b * CHUNK
        pltpu.sync_copy(dst_hbm.at[pl.ds(off, CHUNK)], id_.at[b])
        pltpu.sync_copy(src_hbm.at[pl.ds(off, CHUNK)], is_.at[b])
        pltpu.sync_copy(e_hbm.at[pl.ds(off, CHUNK)], ev_.at[b])
        pltpu.async_copy(h_hbm.at[id_.at[b]], rows2.at[b], sg[b])

    def process(b, c):
        pltpu.make_async_copy(h_hbm.at[id_.at[b]], rows2.at[b], sg[b]).wait()
        off2 = base + (c + 2) * CHUNK

        @pl.when(c + 2 < N_CHUNKS)
        def _():
            pltpu.async_copy(dst_hbm.at[pl.ds(off2, CHUNK)], id_.at[b],
                             sdi[b])

        @pl.when(c >= 2)
        def _():
            pltpu.make_async_copy(
                src_hbm.at[pl.ds(0, CHUNK)], is_.at[b], sis[b]).wait()
            pltpu.make_async_copy(
                e_hbm.at[pl.ds(0, CHUNK)], ev_.at[b], sev[b]).wait()

        for g in range(CHUNK // 16):
            eg = ev_[b, pl.ds(g * 16, 16)]
            for l in range(16):
                r = g * 16 + l
                bc = _bcast_lane(eg, l)
                for j in range(D // 16):
                    sb[r, pl.ds(j * 16, 16)] = (
                        rows2[b, r, pl.ds(j * 16, 16)] * bc)
        pltpu.sync_copy(sb, acc.at[is_.at[b]], add=True)
        pltpu.sync_copy(ev_.at[b], acc_r.at[is_.at[b]], add=True)

        @pl.when(c + 2 < N_CHUNKS)
        def _():
            pltpu.make_async_copy(
                dst_hbm.at[pl.ds(0, CHUNK)], id_.at[b], sdi[b]).wait()
            pltpu.async_copy(h_hbm.at[id_.at[b]], rows2.at[b], sg[b])
            pltpu.async_copy(src_hbm.at[pl.ds(off2, CHUNK)], is_.at[b],
                             sis[b])
            pltpu.async_copy(e_hbm.at[pl.ds(off2, CHUNK)], ev_.at[b],
                             sev[b])

    def step(i, carry):
        for b in range(2):
            process(b, 2 * i + b)
        return carry

    lax.fori_loop(0, (N_CHUNKS - 1) // 2, step, 0)
    process(0, N_CHUNKS - 1)
    plsc.subcore_barrier()
    pltpu.sync_copy(acc.at[pl.ds(r0, ROWS_PER_TILE)],
                    hp_hbm.at[cid, pl.ds(r0, ROWS_PER_TILE)])
    pltpu.sync_copy(acc_r.at[pl.ds(r0, ROWS_PER_TILE)],
                    rs_hbm.at[cid, pl.ds(r0, ROWS_PER_TILE)])


def _fin_body(hp_ref, rs_ref, o_ref):
    hp = hp_ref[0] + hp_ref[1]
    rs = rs_ref[0] + rs_ref[1]
    rs = rs + (rs == 0.0).astype(jnp.float32)
    v = hp / rs
    o_ref[...] = jnp.maximum(v, 0.2 * v)


_fin = pl.pallas_call(
    _fin_body,
    out_shape=jax.ShapeDtypeStruct((N_PAD, D), jnp.float32),
)


def kernel(inputs, edge_index, w, a1_w, a1_b, bn1_g, bn1_b,
           a2_w, a2_b, bn2_g, bn2_b, a3_w, a3_b):
    ei = edge_index.astype(jnp.int32)
    src = ei[0]
    dst = ei[1]
    src3 = src.reshape(NW, N_CHUNKS, CHUNK)
    dst3 = dst.reshape(NW, N_CHUNKS, CHUNK)
    h = _mm_h(inputs, w)
    eh = _sc_edge_diff(h, src3, dst3)
    x1, st1 = _stage1(eh, a1_w, a1_b.reshape(1, D1))
    x2, st2 = _stage2(x1, st1, bn1_g.reshape(1, D1), bn1_b.reshape(1, D1),
                      a2_w, a2_b.reshape(1, D2))
    e = _stage3(x2, st2, bn2_g.reshape(1, D2), bn2_b.reshape(1, D2),
                a3_w.reshape(1, D2), a3_b.reshape(1, 1),
                src.reshape(GE, 1, BE), dst.reshape(GE, 1, BE))
    hp2, rs2 = _sc_scatter(h, src, dst, e.reshape(E))
    return _fin(hp2, rs2.reshape(NC, N_PAD, 1))[:N]


# speedup vs baseline: 3.9583x; 1.2205x over previous
import functools

import jax
import jax.numpy as jnp
import numpy as np
from jax import lax
from jax.experimental import pallas as pl
from jax.experimental.pallas import tpu as pltpu
from jax.experimental.pallas import tpu_sc as plsc

N = 10000
E = 320000
D = 128
D1 = 64
D2 = 32

NC = 2
NS = 16
NW = NC * NS

E_PER_W = E // NW
E_PER_CW = E // NC // NS
CHUNK = 80
N_CHUNKS = E_PER_W // CHUNK
N_PAD = 10240
ROWS_PER_TILE = N_PAD // NS

BE = 2560
GE = E // BE

_mesh = plsc.VectorSubcoreMesh(core_axis_name="c", subcore_axis_name="s")

_GATHER_DNUMS = lax.GatherDimensionNumbers(
    offset_dims=(), collapsed_slice_dims=(0,), start_index_map=(0,))


def _bcast_lane(vec16, lane):
    idx = jnp.full((16, 1), lane, jnp.int32)
    return lax.gather(vec16, idx, _GATHER_DNUMS, (1,),
                      mode=lax.GatherScatterMode.PROMISE_IN_BOUNDS)


def _mm_h_body(x_ref, w_ref, o_ref):
    o_ref[...] = jnp.dot(x_ref[...], w_ref[...],
                         preferred_element_type=jnp.float32)


_mm_h = pl.pallas_call(
    _mm_h_body,
    out_shape=jax.ShapeDtypeStruct((N, D), jnp.float32),
)


@functools.partial(
    pl.kernel,
    mesh=_mesh,
    out_type=jax.ShapeDtypeStruct((E, D), jnp.float32),
    scratch_types=[
        pltpu.VMEM((N_CHUNKS, CHUNK), jnp.int32),
        pltpu.VMEM((N_CHUNKS, CHUNK), jnp.int32),
        pltpu.VMEM((2, CHUNK, D), jnp.float32),
        pltpu.VMEM((2, CHUNK, D), jnp.float32),
        pltpu.VMEM((2, CHUNK, D), jnp.float32),
        pltpu.SemaphoreType.DMA,
        pltpu.SemaphoreType.DMA,
        pltpu.SemaphoreType.DMA,
        pltpu.SemaphoreType.DMA,
        pltpu.SemaphoreType.DMA,
        pltpu.SemaphoreType.DMA,
    ],
)
def _sc_edge_diff(h_hbm, src3_hbm, dst3_hbm, eh_hbm, si2, di2, hs2, hd2, ob2,
                  sga0, sga1, sgb0, sgb1, sst0, sst1):
    cid = lax.axis_index("c")
    sid = lax.axis_index("s")
    wid = sid * NC + cid
    base = wid * E_PER_W
    sga = (sga0, sga1)
    sgb = (sgb0, sgb1)
    sst = (sst0, sst1)

    pltpu.sync_copy(src3_hbm.at[wid], si2)
    pltpu.sync_copy(dst3_hbm.at[wid], di2)
    for b in range(2):
        pltpu.async_copy(h_hbm.at[si2.at[b]], hs2.at[b], sga[b])
        pltpu.async_copy(h_hbm.at[di2.at[b]], hd2.at[b], sgb[b])

    def compute(b, c):
        def row(r, carry2):
            for j in range(D // 16):
                a = hs2[b, r, pl.ds(j * 16, 16)]
                bb = hd2[b, r, pl.ds(j * 16, 16)]
                ob2[b, r, pl.ds(j * 16, 16)] = jnp.abs(a - bb)
            return carry2

        lax.fori_loop(0, CHUNK, row, 0)

    def process(b, c):
        @pl.when(c >= 2)
        def _():
            pltpu.make_async_copy(
                ob2.at[b], eh_hbm.at[pl.ds(0, CHUNK)], sst[b]).wait()
        pltpu.make_async_copy(h_hbm.at[si2.at[b]], hs2.at[b], sga[b]).wait()
        pltpu.make_async_copy(h_hbm.at[di2.at[b]], hd2.at[b], sgb[b]).wait()
        compute(b, c)
        pltpu.async_copy(ob2.at[b], eh_hbm.at[pl.ds(base + c * CHUNK, CHUNK)],
                         sst[b])

        @pl.when(c + 2 < N_CHUNKS)
        def _():
            pltpu.async_copy(h_hbm.at[si2.at[c + 2]], hs2.at[b], sga[b])
            pltpu.async_copy(h_hbm.at[di2.at[c + 2]], hd2.at[b], sgb[b])

    def step(i, carry):
        for b in range(2):
            process(b, 2 * i + b)
        return carry

    lax.fori_loop(0, (N_CHUNKS - 1) // 2, step, 0)
    process(0, N_CHUNKS - 1)
    for b in range(2):
        pltpu.make_async_copy(
            ob2.at[b], eh_hbm.at[pl.ds(0, CHUNK)], sst[b]).wait()


def _stage1_body(eh_ref, a1w_ref, a1b_ref, x1_ref, st_ref):
    x = jnp.dot(eh_ref[...].astype(jnp.bfloat16), a1w_ref[...],
                preferred_element_type=jnp.float32) + a1b_ref[...]
    x1_ref[...] = x.astype(jnp.bfloat16)
    upd = jnp.concatenate(
        [jnp.sum(x, axis=0)[None], jnp.sum(x * x, axis=0)[None],
         jnp.zeros((6, D1), jnp.float32)], axis=0)

    @pl.when(pl.program_id(0) == 0)
    def _():
        st_ref[...] = upd

    @pl.when(pl.program_id(0) > 0)
    def _():
        st_ref[...] = st_ref[...] + upd


_stage1 = pl.pallas_call(
    _stage1_body,
    grid=(GE,),
    in_specs=[
        pl.BlockSpec((BE, D), lambda i: (i, 0)),
        pl.BlockSpec((D, D1), lambda i: (0, 0)),
        pl.BlockSpec((1, D1), lambda i: (0, 0)),
    ],
    out_specs=[
        pl.BlockSpec((BE, D1), lambda i: (i, 0)),
        pl.BlockSpec((8, D1), lambda i: (0, 0)),
    ],
    out_shape=[
        jax.ShapeDtypeStruct((E, D1), jnp.bfloat16),
        jax.ShapeDtypeStruct((8, D1), jnp.float32),
    ],
    compiler_params=pltpu.CompilerParams(dimension_semantics=("arbitrary",)),
)


def _stage2_body(x1_ref, st_ref, g_ref, b_ref, a2w_ref, a2b_ref,
                 x2_ref, st2_ref):
    st = st_ref[...]
    mean = st[0:1, :] * (1.0 / E)
    ex2 = st[1:2, :] * (1.0 / E)
    var = ex2 - mean * mean
    scale = g_ref[...] * lax.rsqrt(var + 1e-5)
    shift = b_ref[...] - mean * scale
    y = x1_ref[...].astype(jnp.float32) * scale + shift
    y = jnp.maximum(y, 0.2 * y)
    x2 = jnp.dot(y.astype(jnp.bfloat16), a2w_ref[...],
                 preferred_element_type=jnp.float32) + a2b_ref[...]
    x2_ref[...] = x2.astype(jnp.bfloat16)
    upd = jnp.concatenate(
        [jnp.sum(x2, axis=0)[None], jnp.sum(x2 * x2, axis=0)[None],
         jnp.zeros((6, D2), jnp.float32)], axis=0)

    @pl.when(pl.program_id(0) == 0)
    def _():
        st2_ref[...] = upd

    @pl.when(pl.program_id(0) > 0)
    def _():
        st2_ref[...] = st2_ref[...] + upd


_stage2 = pl.pallas_call(
    _stage2_body,
    grid=(GE,),
    in_specs=[
        pl.BlockSpec((BE, D1), lambda i: (i, 0)),
        pl.BlockSpec((8, D1), lambda i: (0, 0)),
        pl.BlockSpec((1, D1), lambda i: (0, 0)),
        pl.BlockSpec((1, D1), lambda i: (0, 0)),
        pl.BlockSpec((D1, D2), lambda i: (0, 0)),
        pl.BlockSpec((1, D2), lambda i: (0, 0)),
    ],
    out_specs=[
        pl.BlockSpec((BE, D2), lambda i: (i, 0)),
        pl.BlockSpec((8, D2), lambda i: (0, 0)),
    ],
    out_shape=[
        jax.ShapeDtypeStruct((E, D2), jnp.bfloat16),
        jax.ShapeDtypeStruct((8, D2), jnp.float32),
    ],
    compiler_params=pltpu.CompilerParams(dimension_semantics=("arbitrary",)),
)


def _stage3_body(x2_ref, st2_ref, g_ref, b_ref, a3w_ref, a3b_ref,
                 src_ref, dst_ref, e_ref):
    st = st2_ref[...]
    mean = st[0:1, :] * (1.0 / E)
    ex2 = st[1:2, :] * (1.0 / E)
    var = ex2 - mean * mean
    scale = g_ref[...] * lax.rsqrt(var + 1e-5)
    shift = b_ref[...] - mean * scale
    z = x2_ref[...].astype(jnp.float32) * scale + shift
    z = jnp.maximum(z, 0.2 * z)
    t = lax.dot_general(a3w_ref[...], z, (((1,), (1,)), ((), ())),
                        preferred_element_type=jnp.float32) + a3b_ref[...]
    t = jnp.maximum(t, 0.2 * t)
    ev = jnp.exp(-t)
    ev = ev + (src_ref[0] == dst_ref[0]).astype(jnp.float32)
    e_ref[0] = ev


_stage3 = pl.pallas_call(
    _stage3_body,
    grid=(GE,),
    in_specs=[
        pl.BlockSpec((BE, D2), lambda i: (i, 0)),
        pl.BlockSpec((8, D2), lambda i: (0, 0)),
        pl.BlockSpec((1, D2), lambda i: (0, 0)),
        pl.BlockSpec((1, D2), lambda i: (0, 0)),
        pl.BlockSpec((1, D2), lambda i: (0, 0)),
        pl.BlockSpec((1, 1), lambda i: (0, 0)),
        pl.BlockSpec((1, 1, BE), lambda i: (i, 0, 0)),
        pl.BlockSpec((1, 1, BE), lambda i: (i, 0, 0)),
    ],
    out_specs=pl.BlockSpec((1, 1, BE), lambda i: (i, 0, 0)),
    out_shape=jax.ShapeDtypeStruct((GE, 1, BE), jnp.float32),
    compiler_params=pltpu.CompilerParams(dimension_semantics=("arbitrary",)),
)


@functools.partial(
    pl.kernel,
    mesh=_mesh,
    out_type=[
        jax.ShapeDtypeStruct((NC, N_PAD, D), jnp.float32),
        jax.ShapeDtypeStruct((NC, N_PAD), jnp.float32),
    ],
    scratch_types=[
        pltpu.VMEM((2, CHUNK), jnp.int32),
        pltpu.VMEM((2, CHUNK), jnp.int32),
        pltpu.VMEM((2, CHUNK), jnp.float32),
        pltpu.VMEM((2, CHUNK, D), jnp.float32),
        pltpu.VMEM((CHUNK, D), jnp.float32),
        pltpu.VMEM((ROWS_PER_TILE,), jnp.float32),
        pltpu.VMEM_SHARED((N_PAD, D), jnp.float32),
        pltpu.VMEM_SHARED((N_PAD,), jnp.float32),
        pltpu.SemaphoreType.DMA,
        pltpu.SemaphoreType.DMA,
        pltpu.SemaphoreType.DMA,
        pltpu.SemaphoreType.DMA,
        pltpu.SemaphoreType.DMA,
        pltpu.SemaphoreType.DMA,
        pltpu.SemaphoreType.DMA,
        pltpu.SemaphoreType.DMA,
    ],
)
def _sc_scatter(h_hbm, src_hbm, dst_hbm, e_hbm, hp_hbm, rs_hbm,
                is_, id_, ev_, rows2, sb, zb1, acc, acc_r,
                sg0, sg1, sid0, sid1, sis0, sis1, sev0, sev1):
    cid = lax.axis_index("c")
    sid = lax.axis_index("s")
    sg = (sg0, sg1)
    sdi = (sid0, sid1)
    sis = (sis0, sis1)
    sev = (sev0, sev1)

    def zrow(r, c2):
        for j in range(D // 16):
            sb[r, pl.ds(j * 16, 16)] = jnp.zeros((16,), jnp.float32)
        return c2

    lax.fori_loop(0, CHUNK, zrow, 0)

    def zrow1(r, c2):
        zb1[pl.ds(r * 16, 16)] = jnp.zeros((16,), jnp.float32)
        return c2

    lax.fori_loop(0, ROWS_PER_TILE // 16, zrow1, 0)
    r0 = sid * ROWS_PER_TILE
    for k in range(ROWS_PER_TILE // CHUNK):
        pltpu.sync_copy(sb, acc.at[pl.ds(r0 + k * CHUNK, CHUNK)])
    pltpu.sync_copy(zb1, acc_r.at[pl.ds(r0, ROWS_PER_TILE)])
    plsc.subcore_barrier()

    base = (cid * NS + sid) * E_PER_CW
    for b in range(2):
        off = base + b * CHUNK
        pltpu.sync_copy(dst_hbm.at[pl.ds(off, CHUNK)], id_.at[b])
        pltpu.sync_copy(src_hbm.at[pl.ds(off, CHUNK)], is_.at[b])
        pltpu.sync_copy(e_hbm.at[pl.ds(off, CHUNK)], ev_.at[b])
        pltpu.async_copy(h_hbm.at[id_.at[b]], rows2.at[b], sg[b])

    def process(b, c):
        pltpu.make_async_copy(h_hbm.at[id_.at[b]], rows2.at[b], sg[b]).wait()
        off2 = base + (c + 2) * CHUNK

        @pl.when(c + 2 < N_CHUNKS)
        def _():
            pltpu.async_copy(dst_hbm.at[pl.ds(off2, CHUNK)], id_.at[b],
                             sdi[b])

        @pl.when(c >= 2)
        def _():
            pltpu.make_async_copy(
                src_hbm.at[pl.ds(0, CHUNK)], is_.at[b], sis[b]).wait()
            pltpu.make_async_copy(
                e_hbm.at[pl.ds(0, CHUNK)], ev_.at[b], sev[b]).wait()

        for g in range(CHUNK // 16):
            eg = ev_[b, pl.ds(g * 16, 16)]
            for l in range(16):
                r = g * 16 + l
                bc = _bcast_lane(eg, l)
                for j in range(D // 16):
                    sb[r, pl.ds(j * 16, 16)] = (
                        rows2[b, r, pl.ds(j * 16, 16)] * bc)
        pltpu.sync_copy(sb, acc.at[is_.at[b]], add=True)
        pltpu.sync_copy(ev_.at[b], acc_r.at[is_.at[b]], add=True)

        @pl.when(c + 2 < N_CHUNKS)
        def _():
            pltpu.make_async_copy(
                dst_hbm.at[pl.ds(0, CHUNK)], id_.at[b], sdi[b]).wait()
            pltpu.async_copy(h_hbm.at[id_.at[b]], rows2.at[b], sg[b])
            pltpu.async_copy(src_hbm.at[pl.ds(off2, CHUNK)], is_.at[b],
                             sis[b])
            pltpu.async_copy(e_hbm.at[pl.ds(off2, CHUNK)], ev_.at[b],
                             sev[b])

    def step(i, carry):
        for b in range(2):
            process(b, 2 * i + b)
        return carry

    lax.fori_loop(0, (N_CHUNKS - 1) // 2, step, 0)
    process(0, N_CHUNKS - 1)
    plsc.subcore_barrier()
    pltpu.sync_copy(acc.at[pl.ds(r0, ROWS_PER_TILE)],
                    hp_hbm.at[cid, pl.ds(r0, ROWS_PER_TILE)])
    pltpu.sync_copy(acc_r.at[pl.ds(r0, ROWS_PER_TILE)],
                    rs_hbm.at[cid, pl.ds(r0, ROWS_PER_TILE)])


def _fin_body(hp_ref, rs_ref, o_ref):
    hp = hp_ref[0] + hp_ref[1]
    rs = rs_ref[0] + rs_ref[1]
    rs = rs + (rs == 0.0).astype(jnp.float32)
    v = hp / rs
    o_ref[...] = jnp.maximum(v, 0.2 * v)


_fin = pl.pallas_call(
    _fin_body,
    out_shape=jax.ShapeDtypeStruct((N_PAD, D), jnp.float32),
)


def kernel(inputs, edge_index, w, a1_w, a1_b, bn1_g, bn1_b,
           a2_w, a2_b, bn2_g, bn2_b, a3_w, a3_b):
    ei = edge_index.astype(jnp.int32)
    src = ei[0]
    dst = ei[1]
    src3 = src.reshape(NW, N_CHUNKS, CHUNK)
    dst3 = dst.reshape(NW, N_CHUNKS, CHUNK)
    h = _mm_h(inputs, w)
    eh = _sc_edge_diff(h, src3, dst3)
    x1, st1 = _stage1(eh, a1_w.astype(jnp.bfloat16), a1_b.reshape(1, D1))
    x2, st2 = _stage2(x1, st1, bn1_g.reshape(1, D1), bn1_b.reshape(1, D1),
                      a2_w.astype(jnp.bfloat16), a2_b.reshape(1, D2))
    e = _stage3(x2, st2, bn2_g.reshape(1, D2), bn2_b.reshape(1, D2),
                a3_w.reshape(1, D2), a3_b.reshape(1, 1),
                src.reshape(GE, 1, BE), dst.reshape(GE, 1, BE))
    hp2, rs2 = _sc_scatter(h, src, dst, e.reshape(E))
    return _fin(hp2, rs2.reshape(NC, N_PAD, 1))[:N]


# speedup vs baseline: 5.0475x; 1.2752x over previous
import functools

import jax
import jax.numpy as jnp
import numpy as np
from jax import lax
from jax.experimental import pallas as pl
from jax.experimental.pallas import tpu as pltpu
from jax.experimental.pallas import tpu_sc as plsc

N = 10000
E = 320000
D = 128
D1 = 64
D2 = 32

NC = 2
NS = 16
NW = NC * NS

E_PER_W = E // NW
E_PER_CW = E // NC // NS
CHUNK = 80
N_CHUNKS = E_PER_W // CHUNK
N_PAD = 10240
ROWS_PER_TILE = N_PAD // NS

BE = 16000
GE = E // BE

_mesh = plsc.VectorSubcoreMesh(core_axis_name="c", subcore_axis_name="s")

_GATHER_DNUMS = lax.GatherDimensionNumbers(
    offset_dims=(), collapsed_slice_dims=(0,), start_index_map=(0,))


def _bcast_lane(vec16, lane):
    idx = jnp.full((16, 1), lane, jnp.int32)
    return lax.gather(vec16, idx, _GATHER_DNUMS, (1,),
                      mode=lax.GatherScatterMode.PROMISE_IN_BOUNDS)


def _mm_h_body(x_ref, w_ref, o_ref):
    o_ref[...] = jnp.dot(x_ref[...], w_ref[...],
                         preferred_element_type=jnp.float32)


_mm_h = pl.pallas_call(
    _mm_h_body,
    out_shape=jax.ShapeDtypeStruct((N, D), jnp.float32),
)


@functools.partial(
    pl.kernel,
    mesh=_mesh,
    out_type=jax.ShapeDtypeStruct((E, D), jnp.float32),
    scratch_types=[
        pltpu.VMEM((N_CHUNKS, CHUNK), jnp.int32),
        pltpu.VMEM((N_CHUNKS, CHUNK), jnp.int32),
        pltpu.VMEM((2, CHUNK, D), jnp.float32),
        pltpu.VMEM((2, CHUNK, D), jnp.float32),
        pltpu.VMEM((2, CHUNK, D), jnp.float32),
        pltpu.SemaphoreType.DMA,
        pltpu.SemaphoreType.DMA,
        pltpu.SemaphoreType.DMA,
        pltpu.SemaphoreType.DMA,
        pltpu.SemaphoreType.DMA,
        pltpu.SemaphoreType.DMA,
    ],
)
def _sc_edge_diff(h_hbm, src3_hbm, dst3_hbm, eh_hbm, si2, di2, hs2, hd2, ob2,
                  sga0, sga1, sgb0, sgb1, sst0, sst1):
    cid = lax.axis_index("c")
    sid = lax.axis_index("s")
    wid = sid * NC + cid
    base = wid * E_PER_W
    sga = (sga0, sga1)
    sgb = (sgb0, sgb1)
    sst = (sst0, sst1)

    pltpu.sync_copy(src3_hbm.at[wid], si2)
    pltpu.sync_copy(dst3_hbm.at[wid], di2)
    for b in range(2):
        pltpu.async_copy(h_hbm.at[si2.at[b]], hs2.at[b], sga[b])
        pltpu.async_copy(h_hbm.at[di2.at[b]], hd2.at[b], sgb[b])

    def compute(b, c):
        def row(r, carry2):
            for j in range(D // 16):
                a = hs2[b, r, pl.ds(j * 16, 16)]
                bb = hd2[b, r, pl.ds(j * 16, 16)]
                ob2[b, r, pl.ds(j * 16, 16)] = jnp.abs(a - bb)
            return carry2

        lax.fori_loop(0, CHUNK, row, 0)

    def process(b, c):
        @pl.when(c >= 2)
        def _():
            pltpu.make_async_copy(
                ob2.at[b], eh_hbm.at[pl.ds(0, CHUNK)], sst[b]).wait()
        pltpu.make_async_copy(h_hbm.at[si2.at[b]], hs2.at[b], sga[b]).wait()
        pltpu.make_async_copy(h_hbm.at[di2.at[b]], hd2.at[b], sgb[b]).wait()
        compute(b, c)
        pltpu.async_copy(ob2.at[b], eh_hbm.at[pl.ds(base + c * CHUNK, CHUNK)],
                         sst[b])

        @pl.when(c + 2 < N_CHUNKS)
        def _():
            pltpu.async_copy(h_hbm.at[si2.at[c + 2]], hs2.at[b], sga[b])
            pltpu.async_copy(h_hbm.at[di2.at[c + 2]], hd2.at[b], sgb[b])

    def step(i, carry):
        for b in range(2):
            process(b, 2 * i + b)
        return carry

    lax.fori_loop(0, (N_CHUNKS - 1) // 2, step, 0)
    process(0, N_CHUNKS - 1)
    for b in range(2):
        pltpu.make_async_copy(
            ob2.at[b], eh_hbm.at[pl.ds(0, CHUNK)], sst[b]).wait()


def _stage1_body(eh_ref, a1w_ref, a1b_ref, x1_ref, st_ref):
    x = jnp.dot(eh_ref[...].astype(jnp.bfloat16), a1w_ref[...],
                preferred_element_type=jnp.float32) + a1b_ref[...]
    x1_ref[...] = x.astype(jnp.bfloat16)
    upd = jnp.concatenate(
        [jnp.sum(x, axis=0)[None], jnp.sum(x * x, axis=0)[None],
         jnp.zeros((6, D1), jnp.float32)], axis=0)

    @pl.when(pl.program_id(0) == 0)
    def _():
        st_ref[...] = upd

    @pl.when(pl.program_id(0) > 0)
    def _():
        st_ref[...] = st_ref[...] + upd


_stage1 = pl.pallas_call(
    _stage1_body,
    grid=(GE,),
    in_specs=[
        pl.BlockSpec((BE, D), lambda i: (i, 0)),
        pl.BlockSpec((D, D1), lambda i: (0, 0)),
        pl.BlockSpec((1, D1), lambda i: (0, 0)),
    ],
    out_specs=[
        pl.BlockSpec((BE, D1), lambda i: (i, 0)),
        pl.BlockSpec((8, D1), lambda i: (0, 0)),
    ],
    out_shape=[
        jax.ShapeDtypeStruct((E, D1), jnp.bfloat16),
        jax.ShapeDtypeStruct((8, D1), jnp.float32),
    ],
    compiler_params=pltpu.CompilerParams(dimension_semantics=("arbitrary",)),
)


def _stage2_body(x1_ref, st_ref, g_ref, b_ref, a2w_ref, a2b_ref,
                 x2_ref, st2_ref):
    st = st_ref[...]
    mean = st[0:1, :] * (1.0 / E)
    ex2 = st[1:2, :] * (1.0 / E)
    var = ex2 - mean * mean
    scale = g_ref[...] * lax.rsqrt(var + 1e-5)
    shift = b_ref[...] - mean * scale
    y = x1_ref[...].astype(jnp.float32) * scale + shift
    y = jnp.maximum(y, 0.2 * y)
    x2 = jnp.dot(y.astype(jnp.bfloat16), a2w_ref[...],
                 preferred_element_type=jnp.float32) + a2b_ref[...]
    x2_ref[...] = x2.astype(jnp.bfloat16)
    upd = jnp.concatenate(
        [jnp.sum(x2, axis=0)[None], jnp.sum(x2 * x2, axis=0)[None],
         jnp.zeros((6, D2), jnp.float32)], axis=0)

    @pl.when(pl.program_id(0) == 0)
    def _():
        st2_ref[...] = upd

    @pl.when(pl.program_id(0) > 0)
    def _():
        st2_ref[...] = st2_ref[...] + upd


_stage2 = pl.pallas_call(
    _stage2_body,
    grid=(GE,),
    in_specs=[
        pl.BlockSpec((BE, D1), lambda i: (i, 0)),
        pl.BlockSpec((8, D1), lambda i: (0, 0)),
        pl.BlockSpec((1, D1), lambda i: (0, 0)),
        pl.BlockSpec((1, D1), lambda i: (0, 0)),
        pl.BlockSpec((D1, D2), lambda i: (0, 0)),
        pl.BlockSpec((1, D2), lambda i: (0, 0)),
    ],
    out_specs=[
        pl.BlockSpec((BE, D2), lambda i: (i, 0)),
        pl.BlockSpec((8, D2), lambda i: (0, 0)),
    ],
    out_shape=[
        jax.ShapeDtypeStruct((E, D2), jnp.bfloat16),
        jax.ShapeDtypeStruct((8, D2), jnp.float32),
    ],
    compiler_params=pltpu.CompilerParams(dimension_semantics=("arbitrary",)),
)


def _stage3_body(x2_ref, st2_ref, g_ref, b_ref, a3w_ref, a3b_ref,
                 src_ref, dst_ref, e_ref):
    st = st2_ref[...]
    mean = st[0:1, :] * (1.0 / E)
    ex2 = st[1:2, :] * (1.0 / E)
    var = ex2 - mean * mean
    scale = g_ref[...] * lax.rsqrt(var + 1e-5)
    shift = b_ref[...] - mean * scale
    z = x2_ref[...].astype(jnp.float32) * scale + shift
    z = jnp.maximum(z, 0.2 * z)
    t = lax.dot_general(a3w_ref[...], z, (((1,), (1,)), ((), ())),
                        preferred_element_type=jnp.float32) + a3b_ref[...]
    t = jnp.maximum(t, 0.2 * t)
    ev = jnp.exp(-t)
    ev = ev + (src_ref[0] == dst_ref[0]).astype(jnp.float32)
    e_ref[0] = ev


_stage3 = pl.pallas_call(
    _stage3_body,
    grid=(GE,),
    in_specs=[
        pl.BlockSpec((BE, D2), lambda i: (i, 0)),
        pl.BlockSpec((8, D2), lambda i: (0, 0)),
        pl.BlockSpec((1, D2), lambda i: (0, 0)),
        pl.BlockSpec((1, D2), lambda i: (0, 0)),
        pl.BlockSpec((1, D2), lambda i: (0, 0)),
        pl.BlockSpec((1, 1), lambda i: (0, 0)),
        pl.BlockSpec((1, 1, BE), lambda i: (i, 0, 0)),
        pl.BlockSpec((1, 1, BE), lambda i: (i, 0, 0)),
    ],
    out_specs=pl.BlockSpec((1, 1, BE), lambda i: (i, 0, 0)),
    out_shape=jax.ShapeDtypeStruct((GE, 1, BE), jnp.float32),
    compiler_params=pltpu.CompilerParams(dimension_semantics=("arbitrary",)),
)


@functools.partial(
    pl.kernel,
    mesh=_mesh,
    out_type=[
        jax.ShapeDtypeStruct((NC, N_PAD, D), jnp.float32),
        jax.ShapeDtypeStruct((NC, N_PAD), jnp.float32),
    ],
    scratch_types=[
        pltpu.VMEM((2, CHUNK), jnp.int32),
        pltpu.VMEM((2, CHUNK), jnp.int32),
        pltpu.VMEM((2, CHUNK), jnp.float32),
        pltpu.VMEM((2, CHUNK, D), jnp.float32),
        pltpu.VMEM((CHUNK, D), jnp.float32),
        pltpu.VMEM((ROWS_PER_TILE,), jnp.float32),
        pltpu.VMEM_SHARED((N_PAD, D), jnp.float32),
        pltpu.VMEM_SHARED((N_PAD,), jnp.float32),
        pltpu.SemaphoreType.DMA,
        pltpu.SemaphoreType.DMA,
        pltpu.SemaphoreType.DMA,
        pltpu.SemaphoreType.DMA,
        pltpu.SemaphoreType.DMA,
        pltpu.SemaphoreType.DMA,
        pltpu.SemaphoreType.DMA,
        pltpu.SemaphoreType.DMA,
    ],
)
def _sc_scatter(h_hbm, src_hbm, dst_hbm, e_hbm, hp_hbm, rs_hbm,
                is_, id_, ev_, rows2, sb, zb1, acc, acc_r,
                sg0, sg1, sid0, sid1, sis0, sis1, sev0, sev1):
    cid = lax.axis_index("c")
    sid = lax.axis_index("s")
    sg = (sg0, sg1)
    sdi = (sid0, sid1)
    sis = (sis0, sis1)
    sev = (sev0, sev1)

    def zrow(r, c2):
        for j in range(D // 16):
            sb[r, pl.ds(j * 16, 16)] = jnp.zeros((16,), jnp.float32)
        return c2

    lax.fori_loop(0, CHUNK, zrow, 0)

    def zrow1(r, c2):
        zb1[pl.ds(r * 16, 16)] = jnp.zeros((16,), jnp.float32)
        return c2

    lax.fori_loop(0, ROWS_PER_TILE // 16, zrow1, 0)
    r0 = sid * ROWS_PER_TILE
    for k in range(ROWS_PER_TILE // CHUNK):
        pltpu.sync_copy(sb, acc.at[pl.ds(r0 + k * CHUNK, CHUNK)])
    pltpu.sync_copy(zb1, acc_r.at[pl.ds(r0, ROWS_PER_TILE)])
    plsc.subcore_barrier()

    base = (cid * NS + sid) * E_PER_CW
    for b in range(2):
        off = base + b * CHUNK
        pltpu.sync_copy(dst_hbm.at[pl.ds(off, CHUNK)], id_.at[b])
        pltpu.sync_copy(src_hbm.at[pl.ds(off, CHUNK)], is_.at[b])
        pltpu.sync_copy(e_hbm.at[pl.ds(off, CHUNK)], ev_.at[b])
        pltpu.async_copy(h_hbm.at[id_.at[b]], rows2.at[b], sg[b])

    def process(b, c):
        pltpu.make_async_copy(h_hbm.at[id_.at[b]], rows2.at[b], sg[b]).wait()
        off2 = base + (c + 2) * CHUNK

        @pl.when(c + 2 < N_CHUNKS)
        def _():
            pltpu.async_copy(dst_hbm.at[pl.ds(off2, CHUNK)], id_.at[b],
                             sdi[b])

        @pl.when(c >= 2)
        def _():
            pltpu.make_async_copy(
                src_hbm.at[pl.ds(0, CHUNK)], is_.at[b], sis[b]).wait()
            pltpu.make_async_copy(
                e_hbm.at[pl.ds(0, CHUNK)], ev_.at[b], sev[b]).wait()

        for g in range(CHUNK // 16):
            eg = ev_[b, pl.ds(g * 16, 16)]
            for l in range(16):
                r = g * 16 + l
                bc = _bcast_lane(eg, l)
                for j in range(D // 16):
                    sb[r, pl.ds(j * 16, 16)] = (
                        rows2[b, r, pl.ds(j * 16, 16)] * bc)
        pltpu.sync_copy(sb, acc.at[is_.at[b]], add=True)
        pltpu.sync_copy(ev_.at[b], acc_r.at[is_.at[b]], add=True)

        @pl.when(c + 2 < N_CHUNKS)
        def _():
            pltpu.make_async_copy(
                dst_hbm.at[pl.ds(0, CHUNK)], id_.at[b], sdi[b]).wait()
            pltpu.async_copy(h_hbm.at[id_.at[b]], rows2.at[b], sg[b])
            pltpu.async_copy(src_hbm.at[pl.ds(off2, CHUNK)], is_.at[b],
                             sis[b])
            pltpu.async_copy(e_hbm.at[pl.ds(off2, CHUNK)], ev_.at[b],
                             sev[b])

    def step(i, carry):
        for b in range(2):
            process(b, 2 * i + b)
        return carry

    lax.fori_loop(0, (N_CHUNKS - 1) // 2, step, 0)
    process(0, N_CHUNKS - 1)
    plsc.subcore_barrier()
    pltpu.sync_copy(acc.at[pl.ds(r0, ROWS_PER_TILE)],
                    hp_hbm.at[cid, pl.ds(r0, ROWS_PER_TILE)])
    pltpu.sync_copy(acc_r.at[pl.ds(r0, ROWS_PER_TILE)],
                    rs_hbm.at[cid, pl.ds(r0, ROWS_PER_TILE)])


def _fin_body(hp_ref, rs_ref, o_ref):
    hp = hp_ref[0] + hp_ref[1]
    rs = rs_ref[0] + rs_ref[1]
    rs = rs + (rs == 0.0).astype(jnp.float32)
    v = hp / rs
    o_ref[...] = jnp.maximum(v, 0.2 * v)


_fin = pl.pallas_call(
    _fin_body,
    out_shape=jax.ShapeDtypeStruct((N_PAD, D), jnp.float32),
)


def kernel(inputs, edge_index, w, a1_w, a1_b, bn1_g, bn1_b,
           a2_w, a2_b, bn2_g, bn2_b, a3_w, a3_b):
    ei = edge_index.astype(jnp.int32)
    src = ei[0]
    dst = ei[1]
    src3 = src.reshape(NW, N_CHUNKS, CHUNK)
    dst3 = dst.reshape(NW, N_CHUNKS, CHUNK)
    h = _mm_h(inputs, w)
    eh = _sc_edge_diff(h, src3, dst3)
    x1, st1 = _stage1(eh, a1_w.astype(jnp.bfloat16), a1_b.reshape(1, D1))
    x2, st2 = _stage2(x1, st1, bn1_g.reshape(1, D1), bn1_b.reshape(1, D1),
                      a2_w.astype(jnp.bfloat16), a2_b.reshape(1, D2))
    e = _stage3(x2, st2, bn2_g.reshape(1, D2), bn2_b.reshape(1, D2),
                a3_w.reshape(1, D2), a3_b.reshape(1, 1),
                src.reshape(GE, 1, BE), dst.reshape(GE, 1, BE))
    hp2, rs2 = _sc_scatter(h, src, dst, e.reshape(E))
    return _fin(hp2, rs2.reshape(NC, N_PAD, 1))[:N]


# speedup vs baseline: 5.1227x; 1.0149x over previous
import functools

import jax
import jax.numpy as jnp
import numpy as np
from jax import lax
from jax.experimental import pallas as pl
from jax.experimental.pallas import tpu as pltpu
from jax.experimental.pallas import tpu_sc as plsc

N = 10000
E = 320000
D = 128
D1 = 64
D2 = 32

NC = 2
NS = 16
NW = NC * NS

E_PER_W = E // NW
E_PER_CW = E // NC // NS
CHUNK = 80
N_CHUNKS = E_PER_W // CHUNK
N_PAD = 10240
ROWS_PER_TILE = N_PAD // NS

BE = 32000
GE = E // BE

_mesh = plsc.VectorSubcoreMesh(core_axis_name="c", subcore_axis_name="s")

_GATHER_DNUMS = lax.GatherDimensionNumbers(
    offset_dims=(), collapsed_slice_dims=(0,), start_index_map=(0,))


def _bcast_lane(vec16, lane):
    idx = jnp.full((16, 1), lane, jnp.int32)
    return lax.gather(vec16, idx, _GATHER_DNUMS, (1,),
                      mode=lax.GatherScatterMode.PROMISE_IN_BOUNDS)


def _mm_h_body(x_ref, w_ref, o_ref):
    o_ref[...] = jnp.dot(x_ref[...], w_ref[...],
                         preferred_element_type=jnp.float32)


_mm_h = pl.pallas_call(
    _mm_h_body,
    out_shape=jax.ShapeDtypeStruct((N, D), jnp.float32),
)


@functools.partial(
    pl.kernel,
    mesh=_mesh,
    out_type=jax.ShapeDtypeStruct((E, D), jnp.float32),
    scratch_types=[
        pltpu.VMEM((N_CHUNKS, CHUNK), jnp.int32),
        pltpu.VMEM((N_CHUNKS, CHUNK), jnp.int32),
        pltpu.VMEM((2, CHUNK, D), jnp.float32),
        pltpu.VMEM((2, CHUNK, D), jnp.float32),
        pltpu.VMEM((2, CHUNK, D), jnp.float32),
        pltpu.SemaphoreType.DMA,
        pltpu.SemaphoreType.DMA,
        pltpu.SemaphoreType.DMA,
        pltpu.SemaphoreType.DMA,
        pltpu.SemaphoreType.DMA,
        pltpu.SemaphoreType.DMA,
    ],
)
def _sc_edge_diff(h_hbm, src3_hbm, dst3_hbm, eh_hbm, si2, di2, hs2, hd2, ob2,
                  sga0, sga1, sgb0, sgb1, sst0, sst1):
    cid = lax.axis_index("c")
    sid = lax.axis_index("s")
    wid = sid * NC + cid
    base = wid * E_PER_W
    sga = (sga0, sga1)
    sgb = (sgb0, sgb1)
    sst = (sst0, sst1)

    pltpu.sync_copy(src3_hbm.at[wid], si2)
    pltpu.sync_copy(dst3_hbm.at[wid], di2)
    for b in range(2):
        pltpu.async_copy(h_hbm.at[si2.at[b]], hs2.at[b], sga[b])
        pltpu.async_copy(h_hbm.at[di2.at[b]], hd2.at[b], sgb[b])

    def compute(b, c):
        def row(r2, carry2):
            for u in range(2):
                r = r2 * 2 + u
                for j in range(D // 16):
                    a = hs2[b, r, pl.ds(j * 16, 16)]
                    bb = hd2[b, r, pl.ds(j * 16, 16)]
                    ob2[b, r, pl.ds(j * 16, 16)] = jnp.abs(a - bb)
            return carry2

        lax.fori_loop(0, CHUNK // 2, row, 0)

    def process(b, c):
        @pl.when(c >= 2)
        def _():
            pltpu.make_async_copy(
                ob2.at[b], eh_hbm.at[pl.ds(0, CHUNK)], sst[b]).wait()
        pltpu.make_async_copy(h_hbm.at[si2.at[b]], hs2.at[b], sga[b]).wait()
        pltpu.make_async_copy(h_hbm.at[di2.at[b]], hd2.at[b], sgb[b]).wait()
        compute(b, c)
        pltpu.async_copy(ob2.at[b], eh_hbm.at[pl.ds(base + c * CHUNK, CHUNK)],
                         sst[b])

        @pl.when(c + 2 < N_CHUNKS)
        def _():
            pltpu.async_copy(h_hbm.at[si2.at[c + 2]], hs2.at[b], sga[b])
            pltpu.async_copy(h_hbm.at[di2.at[c + 2]], hd2.at[b], sgb[b])

    def step(i, carry):
        for b in range(2):
            process(b, 2 * i + b)
        return carry

    lax.fori_loop(0, (N_CHUNKS - 1) // 2, step, 0)
    process(0, N_CHUNKS - 1)
    for b in range(2):
        pltpu.make_async_copy(
            ob2.at[b], eh_hbm.at[pl.ds(0, CHUNK)], sst[b]).wait()


def _stage1_body(eh_ref, a1w_ref, a1b_ref, x1_ref, st_ref):
    x = jnp.dot(eh_ref[...].astype(jnp.bfloat16), a1w_ref[...],
                preferred_element_type=jnp.float32) + a1b_ref[...]
    x1_ref[...] = x.astype(jnp.bfloat16)
    upd = jnp.concatenate(
        [jnp.sum(x, axis=0)[None], jnp.sum(x * x, axis=0)[None],
         jnp.zeros((6, D1), jnp.float32)], axis=0)

    @pl.when(pl.program_id(0) == 0)
    def _():
        st_ref[...] = upd

    @pl.when(pl.program_id(0) > 0)
    def _():
        st_ref[...] = st_ref[...] + upd


_stage1 = pl.pallas_call(
    _stage1_body,
    grid=(GE,),
    in_specs=[
        pl.BlockSpec((BE, D), lambda i: (i, 0)),
        pl.BlockSpec((D, D1), lambda i: (0, 0)),
        pl.BlockSpec((1, D1), lambda i: (0, 0)),
    ],
    out_specs=[
        pl.BlockSpec((BE, D1), lambda i: (i, 0)),
        pl.BlockSpec((8, D1), lambda i: (0, 0)),
    ],
    out_shape=[
        jax.ShapeDtypeStruct((E, D1), jnp.bfloat16),
        jax.ShapeDtypeStruct((8, D1), jnp.float32),
    ],
    compiler_params=pltpu.CompilerParams(dimension_semantics=("arbitrary",)),
)


def _stage2_body(x1_ref, st_ref, g_ref, b_ref, a2w_ref, a2b_ref,
                 x2_ref, st2_ref):
    st = st_ref[...]
    mean = st[0:1, :] * (1.0 / E)
    ex2 = st[1:2, :] * (1.0 / E)
    var = ex2 - mean * mean
    scale = g_ref[...] * lax.rsqrt(var + 1e-5)
    shift = b_ref[...] - mean * scale
    y = x1_ref[...].astype(jnp.float32) * scale + shift
    y = jnp.maximum(y, 0.2 * y)
    x2 = jnp.dot(y.astype(jnp.bfloat16), a2w_ref[...],
                 preferred_element_type=jnp.float32) + a2b_ref[...]
    x2_ref[...] = x2.astype(jnp.bfloat16)
    upd = jnp.concatenate(
        [jnp.sum(x2, axis=0)[None], jnp.sum(x2 * x2, axis=0)[None],
         jnp.zeros((6, D2), jnp.float32)], axis=0)

    @pl.when(pl.program_id(0) == 0)
    def _():
        st2_ref[...] = upd

    @pl.when(pl.program_id(0) > 0)
    def _():
        st2_ref[...] = st2_ref[...] + upd


_stage2 = pl.pallas_call(
    _stage2_body,
    grid=(GE,),
    in_specs=[
        pl.BlockSpec((BE, D1), lambda i: (i, 0)),
        pl.BlockSpec((8, D1), lambda i: (0, 0)),
        pl.BlockSpec((1, D1), lambda i: (0, 0)),
        pl.BlockSpec((1, D1), lambda i: (0, 0)),
        pl.BlockSpec((D1, D2), lambda i: (0, 0)),
        pl.BlockSpec((1, D2), lambda i: (0, 0)),
    ],
    out_specs=[
        pl.BlockSpec((BE, D2), lambda i: (i, 0)),
        pl.BlockSpec((8, D2), lambda i: (0, 0)),
    ],
    out_shape=[
        jax.ShapeDtypeStruct((E, D2), jnp.bfloat16),
        jax.ShapeDtypeStruct((8, D2), jnp.float32),
    ],
    compiler_params=pltpu.CompilerParams(dimension_semantics=("arbitrary",)),
)


def _stage3_body(x2_ref, st2_ref, g_ref, b_ref, a3w_ref, a3b_ref,
                 src_ref, dst_ref, e_ref):
    st = st2_ref[...]
    mean = st[0:1, :] * (1.0 / E)
    ex2 = st[1:2, :] * (1.0 / E)
    var = ex2 - mean * mean
    scale = g_ref[...] * lax.rsqrt(var + 1e-5)
    shift = b_ref[...] - mean * scale
    z = x2_ref[...].astype(jnp.float32) * scale + shift
    z = jnp.maximum(z, 0.2 * z)
    t = lax.dot_general(a3w_ref[...], z, (((1,), (1,)), ((), ())),
                        preferred_element_type=jnp.float32) + a3b_ref[...]
    t = jnp.maximum(t, 0.2 * t)
    ev = jnp.exp(-t)
    ev = ev + (src_ref[0] == dst_ref[0]).astype(jnp.float32)
    e_ref[0] = ev


_stage3 = pl.pallas_call(
    _stage3_body,
    grid=(GE,),
    in_specs=[
        pl.BlockSpec((BE, D2), lambda i: (i, 0)),
        pl.BlockSpec((8, D2), lambda i: (0, 0)),
        pl.BlockSpec((1, D2), lambda i: (0, 0)),
        pl.BlockSpec((1, D2), lambda i: (0, 0)),
        pl.BlockSpec((1, D2), lambda i: (0, 0)),
        pl.BlockSpec((1, 1), lambda i: (0, 0)),
        pl.BlockSpec((1, 1, BE), lambda i: (i, 0, 0)),
        pl.BlockSpec((1, 1, BE), lambda i: (i, 0, 0)),
    ],
    out_specs=pl.BlockSpec((1, 1, BE), lambda i: (i, 0, 0)),
    out_shape=jax.ShapeDtypeStruct((GE, 1, BE), jnp.float32),
    compiler_params=pltpu.CompilerParams(dimension_semantics=("arbitrary",)),
)


@functools.partial(
    pl.kernel,
    mesh=_mesh,
    out_type=[
        jax.ShapeDtypeStruct((NC, N_PAD, D), jnp.float32),
        jax.ShapeDtypeStruct((NC, N_PAD), jnp.float32),
    ],
    scratch_types=[
        pltpu.VMEM((2, CHUNK), jnp.int32),
        pltpu.VMEM((2, CHUNK), jnp.int32),
        pltpu.VMEM((2, CHUNK), jnp.float32),
        pltpu.VMEM((2, CHUNK, D), jnp.float32),
        pltpu.VMEM((CHUNK, D), jnp.float32),
        pltpu.VMEM((ROWS_PER_TILE,), jnp.float32),
        pltpu.VMEM_SHARED((N_PAD, D), jnp.float32),
        pltpu.VMEM_SHARED((N_PAD,), jnp.float32),
        pltpu.SemaphoreType.DMA,
        pltpu.SemaphoreType.DMA,
        pltpu.SemaphoreType.DMA,
        pltpu.SemaphoreType.DMA,
        pltpu.SemaphoreType.DMA,
        pltpu.SemaphoreType.DMA,
        pltpu.SemaphoreType.DMA,
        pltpu.SemaphoreType.DMA,
    ],
)
def _sc_scatter(h_hbm, src_hbm, dst_hbm, e_hbm, hp_hbm, rs_hbm,
                is_, id_, ev_, rows2, sb, zb1, acc, acc_r,
                sg0, sg1, sid0, sid1, sis0, sis1, sev0, sev1):
    cid = lax.axis_index("c")
    sid = lax.axis_index("s")
    sg = (sg0, sg1)
    sdi = (sid0, sid1)
    sis = (sis0, sis1)
    sev = (sev0, sev1)

    def zrow(r, c2):
        for j in range(D // 16):
            sb[r, pl.ds(j * 16, 16)] = jnp.zeros((16,), jnp.float32)
        return c2

    lax.fori_loop(0, CHUNK, zrow, 0)

    def zrow1(r, c2):
        zb1[pl.ds(r * 16, 16)] = jnp.zeros((16,), jnp.float32)
        return c2

    lax.fori_loop(0, ROWS_PER_TILE // 16, zrow1, 0)
    r0 = sid * ROWS_PER_TILE
    for k in range(ROWS_PER_TILE // CHUNK):
        pltpu.sync_copy(sb, acc.at[pl.ds(r0 + k * CHUNK, CHUNK)])
    pltpu.sync_copy(zb1, acc_r.at[pl.ds(r0, ROWS_PER_TILE)])
    plsc.subcore_barrier()

    base = (cid * NS + sid) * E_PER_CW
    for b in range(2):
        off = base + b * CHUNK
        pltpu.sync_copy(dst_hbm.at[pl.ds(off, CHUNK)], id_.at[b])
        pltpu.sync_copy(src_hbm.at[pl.ds(off, CHUNK)], is_.at[b])
        pltpu.sync_copy(e_hbm.at[pl.ds(off, CHUNK)], ev_.at[b])
        pltpu.async_copy(h_hbm.at[id_.at[b]], rows2.at[b], sg[b])

    def process(b, c):
        pltpu.make_async_copy(h_hbm.at[id_.at[b]], rows2.at[b], sg[b]).wait()
        off2 = base + (c + 2) * CHUNK

        @pl.when(c + 2 < N_CHUNKS)
        def _():
            pltpu.async_copy(dst_hbm.at[pl.ds(off2, CHUNK)], id_.at[b],
                             sdi[b])

        @pl.when(c >= 2)
        def _():
            pltpu.make_async_copy(
                src_hbm.at[pl.ds(0, CHUNK)], is_.at[b], sis[b]).wait()
            pltpu.make_async_copy(
                e_hbm.at[pl.ds(0, CHUNK)], ev_.at[b], sev[b]).wait()

        for g in range(CHUNK // 16):
            eg = ev_[b, pl.ds(g * 16, 16)]
            for l in range(16):
                r = g * 16 + l
                bc = _bcast_lane(eg, l)
                for j in range(D // 16):
                    sb[r, pl.ds(j * 16, 16)] = (
                        rows2[b, r, pl.ds(j * 16, 16)] * bc)
        pltpu.sync_copy(sb, acc.at[is_.at[b]], add=True)
        pltpu.sync_copy(ev_.at[b], acc_r.at[is_.at[b]], add=True)

        @pl.when(c + 2 < N_CHUNKS)
        def _():
            pltpu.make_async_copy(
                dst_hbm.at[pl.ds(0, CHUNK)], id_.at[b], sdi[b]).wait()
            pltpu.async_copy(h_hbm.at[id_.at[b]], rows2.at[b], sg[b])
            pltpu.async_copy(src_hbm.at[pl.ds(off2, CHUNK)], is_.at[b],
                             sis[b])
            pltpu.async_copy(e_hbm.at[pl.ds(off2, CHUNK)], ev_.at[b],
                             sev[b])

    def step(i, carry):
        for b in range(2):
            process(b, 2 * i + b)
        return carry

    lax.fori_loop(0, (N_CHUNKS - 1) // 2, step, 0)
    process(0, N_CHUNKS - 1)
    plsc.subcore_barrier()
    pltpu.sync_copy(acc.at[pl.ds(r0, ROWS_PER_TILE)],
                    hp_hbm.at[cid, pl.ds(r0, ROWS_PER_TILE)])
    pltpu.sync_copy(acc_r.at[pl.ds(r0, ROWS_PER_TILE)],
                    rs_hbm.at[cid, pl.ds(r0, ROWS_PER_TILE)])


def _fin_body(hp_ref, rs_ref, o_ref):
    hp = hp_ref[0] + hp_ref[1]
    rs = rs_ref[0] + rs_ref[1]
    rs = rs + (rs == 0.0).astype(jnp.float32)
    v = hp / rs
    o_ref[...] = jnp.maximum(v, 0.2 * v)


_fin = pl.pallas_call(
    _fin_body,
    out_shape=jax.ShapeDtypeStruct((N_PAD, D), jnp.float32),
)


def kernel(inputs, edge_index, w, a1_w, a1_b, bn1_g, bn1_b,
           a2_w, a2_b, bn2_g, bn2_b, a3_w, a3_b):
    ei = edge_index.astype(jnp.int32)
    src = ei[0]
    dst = ei[1]
    src3 = src.reshape(NW, N_CHUNKS, CHUNK)
    dst3 = dst.reshape(NW, N_CHUNKS, CHUNK)
    h = _mm_h(inputs, w)
    eh = _sc_edge_diff(h, src3, dst3)
    x1, st1 = _stage1(eh, a1_w.astype(jnp.bfloat16), a1_b.reshape(1, D1))
    x2, st2 = _stage2(x1, st1, bn1_g.reshape(1, D1), bn1_b.reshape(1, D1),
                      a2_w.astype(jnp.bfloat16), a2_b.reshape(1, D2))
    e = _stage3(x2, st2, bn2_g.reshape(1, D2), bn2_b.reshape(1, D2),
                a3_w.reshape(1, D2), a3_b.reshape(1, 1),
                src.reshape(GE, 1, BE), dst.reshape(GE, 1, BE))
    hp2, rs2 = _sc_scatter(h, src, dst, e.reshape(E))
    return _fin(hp2, rs2.reshape(NC, N_PAD, 1))[:N]
